# baseline (device time: 28954 ns/iter reference)
import jax
import jax.numpy as jnp
from jax import lax
from jax.experimental import pallas as pl
from jax.experimental.pallas import tpu as pltpu

N_DEV = 4
B_LOC = 2
SQ = 256
SKV = 256
HQ = 16
H_PER = HQ // N_DEV
DH = 64
DM = 512
DG = H_PER * DH


def kernel(x, Wq, K_ext, V_ext, Wo):
    my = lax.axis_index("i")

    x_b = x.astype(jnp.bfloat16)
    wq_b = Wq.astype(jnp.bfloat16)
    wo_b = Wo.astype(jnp.bfloat16)
    k_loc = lax.dynamic_slice_in_dim(K_ext, my * B_LOC, B_LOC, 0)
    v_loc = lax.dynamic_slice_in_dim(V_ext, my * B_LOC, B_LOC, 0)
    k_b = k_loc.astype(jnp.bfloat16).transpose(0, 2, 1, 3)
    v_b = v_loc.astype(jnp.bfloat16).transpose(0, 2, 1, 3)

    def body(x_ref, wq_ref, k_ref, v_ref, wo_ref, out_ref,
             wq_g, wo_g, send_sems, recv_sems):
        my_i = lax.axis_index("i")

        bar = pltpu.get_barrier_semaphore()
        for off in (1, 2, 3):
            peer = lax.rem(my_i + off, N_DEV)
            pl.semaphore_signal(bar, inc=1, device_id=(peer,),
                                device_id_type=pl.DeviceIdType.MESH)
        pl.semaphore_wait(bar, N_DEV - 1)

        sends = []
        for idx, off in enumerate((1, 2, 3)):
            peer = lax.rem(my_i + off, N_DEV)
            rq = pltpu.make_async_remote_copy(
                src_ref=wq_ref, dst_ref=wq_g.at[my_i],
                send_sem=send_sems.at[idx, 0], recv_sem=recv_sems.at[my_i, 0],
                device_id=(peer,), device_id_type=pl.DeviceIdType.MESH)
            ro = pltpu.make_async_remote_copy(
                src_ref=wo_ref, dst_ref=wo_g.at[my_i],
                send_sem=send_sems.at[idx, 1], recv_sem=recv_sems.at[my_i, 1],
                device_id=(peer,), device_id_type=pl.DeviceIdType.MESH)
            rq.start()
            ro.start()
            sends.append((rq, ro))

        qi = lax.broadcasted_iota(jnp.int32, (SQ, SKV), 0)
        ki = lax.broadcasted_iota(jnp.int32, (SQ, SKV), 1)
        mask = (jnp.abs(qi - ki) <= 128) | (ki < 32) | (qi < 32)

        def compute_group(g, wq_mat, wo_mat, first):
            for b in range(B_LOC):
                xv = x_ref[b]
                q = jnp.dot(xv, wq_mat,
                            preferred_element_type=jnp.float32)
                qb = q.astype(jnp.bfloat16)
                ctxs = []
                for h in range(H_PER):
                    head = g * H_PER + h
                    kh = k_ref[b, head]
                    vh = v_ref[b, head]
                    qh = qb[:, h * DH:(h + 1) * DH]
                    s = lax.dot_general(
                        qh, kh, (((1,), (1,)), ((), ())),
                        preferred_element_type=jnp.float32) * 0.125
                    s = jnp.where(mask, s, -1e9)
                    m = jnp.max(s, axis=1, keepdims=True)
                    w = jnp.exp(s - m)
                    w = w / jnp.sum(w, axis=1, keepdims=True)
                    ctxs.append(jnp.dot(w.astype(jnp.bfloat16), vh,
                                        preferred_element_type=jnp.float32))
                ctx = jnp.concatenate(ctxs, axis=1).astype(jnp.bfloat16)
                part = jnp.dot(ctx, wo_mat,
                               preferred_element_type=jnp.float32)
                if first:
                    out_ref[b] = part
                else:
                    out_ref[b] = out_ref[b] + part

        compute_group(my_i, wq_ref[...], wo_ref[...], first=True)

        for off in (1, 2, 3):
            src = lax.rem(my_i - off + N_DEV, N_DEV)
            rq = pltpu.make_async_remote_copy(
                src_ref=wq_ref, dst_ref=wq_g.at[src],
                send_sem=send_sems.at[0, 0], recv_sem=recv_sems.at[src, 0],
                device_id=(src,), device_id_type=pl.DeviceIdType.MESH)
            ro = pltpu.make_async_remote_copy(
                src_ref=wo_ref, dst_ref=wo_g.at[src],
                send_sem=send_sems.at[0, 1], recv_sem=recv_sems.at[src, 1],
                device_id=(src,), device_id_type=pl.DeviceIdType.MESH)
            rq.wait_recv()
            ro.wait_recv()
            compute_group(src, wq_g[src], wo_g[src], first=False)

        for rq, ro in sends:
            rq.wait_send()
            ro.wait_send()

    return pl.pallas_call(
        body,
        out_shape=jax.ShapeDtypeStruct((B_LOC, SQ, DM), jnp.float32),
        in_specs=[pl.BlockSpec(memory_space=pltpu.VMEM)] * 5,
        out_specs=pl.BlockSpec(memory_space=pltpu.VMEM),
        scratch_shapes=[
            pltpu.VMEM((N_DEV, DM, DG), jnp.bfloat16),
            pltpu.VMEM((N_DEV, DG, DM), jnp.bfloat16),
            pltpu.SemaphoreType.DMA((3, 2)),
            pltpu.SemaphoreType.DMA((N_DEV, 2)),
        ],
        compiler_params=pltpu.CompilerParams(collective_id=0),
    )(x_b, wq_b, k_b, v_b, wo_b)
